# baseline (speedup 1.0000x reference)
import functools
import math

import jax
import jax.numpy as jnp
from jax import lax
from jax.experimental import pallas as pl
from jax.experimental.pallas import tpu as pltpu
from jax.experimental.pallas import tpu_sc as plsc

VOCAB = 100000
DMODEL = 768
MAXCTX = 2048
BATCH = 4
SEQ = 2048

NUM_CORES = 2
NUM_SUBCORES = 16
NW = NUM_CORES * NUM_SUBCORES
POS_PER_W = SEQ // NW
CHUNK = 8
NCHUNK = POS_PER_W // CHUNK
ROWS = BATCH * CHUNK
LANES = 16
VREGS_PER_ROW = DMODEL // LANES
SCALE = math.sqrt(float(DMODEL))


def _positional_encoding():
    pos = jnp.arange(MAXCTX, dtype=jnp.float32)[:, None]
    i = jnp.arange(DMODEL, dtype=jnp.float32)[None, :]
    angles = 1.0 / jnp.power(10000.0, 2.0 * jnp.floor(i / 2.0) / jnp.float32(DMODEL))
    angle_rads = pos * angles
    sines = jnp.sin(angle_rads[:, 0::2])
    cosines = jnp.cos(angle_rads[:, 1::2])
    return jnp.concatenate([sines, cosines], axis=-1).astype(jnp.float32)


_mesh = plsc.VectorSubcoreMesh(core_axis_name="c", subcore_axis_name="s")


@functools.partial(
    pl.kernel,
    mesh=_mesh,
    out_type=jax.ShapeDtypeStruct((BATCH, SEQ, DMODEL), jnp.float32),
    scratch_types=[
        pltpu.VMEM((NCHUNK, ROWS), jnp.int32),
        pltpu.VMEM((POS_PER_W, DMODEL), jnp.float32),
        pltpu.VMEM((ROWS, DMODEL), jnp.float32),
        pltpu.SemaphoreType.DMA,
    ],
)
def _sc_embed(xr_hbm, table_hbm, pe_hbm, out_hbm, idx_v, pe_v, rows_v, sem):
    w = lax.axis_index("s") * NUM_CORES + lax.axis_index("c")
    pos0 = w * POS_PER_W

    pltpu.sync_copy(xr_hbm.at[w], idx_v)
    pltpu.sync_copy(pe_hbm.at[pl.ds(pos0, POS_PER_W)], pe_v)

    for k in range(NCHUNK):
        pltpu.async_copy(table_hbm.at[idx_v.at[k]], rows_v, sem).wait()

        def body_i(i, _):
            def body_c(c, _):
                pv = pe_v[k * CHUNK + i, pl.ds(c * LANES, LANES)]
                for b in range(BATCH):
                    r = rows_v[b * CHUNK + i, pl.ds(c * LANES, LANES)]
                    rows_v[b * CHUNK + i, pl.ds(c * LANES, LANES)] = r * SCALE + pv
                return 0

            lax.fori_loop(0, VREGS_PER_ROW, body_c, 0)
            return 0

        lax.fori_loop(0, CHUNK, body_i, 0)

        for b in range(BATCH):
            pltpu.sync_copy(
                rows_v.at[pl.ds(b * CHUNK, CHUNK)],
                out_hbm.at[b, pl.ds(pos0 + k * CHUNK, CHUNK)],
            )


def kernel(x, table):
    x = x.astype(jnp.int32)
    xr = x.reshape(BATCH, NW, NCHUNK, CHUNK).transpose(1, 2, 0, 3)
    xr = xr.reshape(NW, NCHUNK, ROWS)
    pe = _positional_encoding()
    return _sc_embed(xr, table, pe)


# speedup vs baseline: 1.0608x; 1.0608x over previous
import functools
import math

import jax
import jax.numpy as jnp
from jax import lax
from jax.experimental import pallas as pl
from jax.experimental.pallas import tpu as pltpu
from jax.experimental.pallas import tpu_sc as plsc

VOCAB = 100000
DMODEL = 768
MAXCTX = 2048
BATCH = 4
SEQ = 2048

NUM_CORES = 2
NUM_SUBCORES = 16
NW = NUM_CORES * NUM_SUBCORES
POS_PER_W = SEQ // NW
CHUNK = 8
NCHUNK = POS_PER_W // CHUNK
ROWS = BATCH * CHUNK
NSLOT = 3
LANES = 16
VREGS_PER_ROW = DMODEL // LANES
UNROLL = 8
SCALE = math.sqrt(float(DMODEL))


def _positional_encoding():
    pos = jnp.arange(MAXCTX, dtype=jnp.float32)[:, None]
    i = jnp.arange(DMODEL, dtype=jnp.float32)[None, :]
    angles = 1.0 / jnp.power(10000.0, 2.0 * jnp.floor(i / 2.0) / jnp.float32(DMODEL))
    angle_rads = pos * angles
    sines = jnp.sin(angle_rads[:, 0::2])
    cosines = jnp.cos(angle_rads[:, 1::2])
    return jnp.concatenate([sines, cosines], axis=-1).astype(jnp.float32)


_mesh = plsc.VectorSubcoreMesh(core_axis_name="c", subcore_axis_name="s")


@functools.partial(
    pl.kernel,
    mesh=_mesh,
    out_type=jax.ShapeDtypeStruct((BATCH, SEQ, DMODEL), jnp.float32),
    scratch_types=[
        pltpu.VMEM((NCHUNK, ROWS), jnp.int32),
        pltpu.VMEM((POS_PER_W, DMODEL), jnp.float32),
        pltpu.VMEM((ROWS, DMODEL), jnp.float32),
        pltpu.VMEM((ROWS, DMODEL), jnp.float32),
        pltpu.VMEM((ROWS, DMODEL), jnp.float32),
        pltpu.SemaphoreType.DMA,
        pltpu.SemaphoreType.DMA,
        pltpu.SemaphoreType.DMA,
        pltpu.SemaphoreType.DMA,
        pltpu.SemaphoreType.DMA,
        pltpu.SemaphoreType.DMA,
    ],
)
def _sc_embed(xr_hbm, table_hbm, pe_hbm, out_hbm, idx_v, pe_v,
              rows0, rows1, rows2, g0, g1, g2, w0, w1, w2, ):
    rows = (rows0, rows1, rows2)
    gsem = (g0, g1, g2)
    wsem = (w0, w1, w2)
    w = lax.axis_index("s") * NUM_CORES + lax.axis_index("c")
    pos0 = w * POS_PER_W

    pltpu.sync_copy(xr_hbm.at[w], idx_v)
    pltpu.sync_copy(pe_hbm.at[pl.ds(pos0, POS_PER_W)], pe_v)

    def compute(rv, k):
        def body_i(i, _):
            def body_c(c4, _):
                for u in range(UNROLL):
                    start = pl.multiple_of((c4 * UNROLL + u) * LANES, LANES)
                    sl = pl.ds(start, LANES)
                    pv = pe_v[k * CHUNK + i, sl]
                    for b in range(BATCH):
                        r = rv[b * CHUNK + i, sl]
                        rv[b * CHUNK + i, sl] = r * SCALE + pv
                return 0

            lax.fori_loop(0, VREGS_PER_ROW // UNROLL, body_c, 0)
            return 0

        lax.fori_loop(0, CHUNK, body_i, 0)

    gcp = [None] * NSLOT
    wcp = [None] * NSLOT
    gcp[0] = pltpu.async_copy(table_hbm.at[idx_v.at[0]], rows[0], gsem[0])
    for k in range(NCHUNK):
        s = k % NSLOT
        if k + 1 < NCHUNK:
            ns = (k + 1) % NSLOT
            if wcp[ns] is not None:
                for c in wcp[ns]:
                    c.wait()
                wcp[ns] = None
            gcp[ns] = pltpu.async_copy(
                table_hbm.at[idx_v.at[k + 1]], rows[ns], gsem[ns])
        gcp[s].wait()
        compute(rows[s], k)
        wcp[s] = [
            pltpu.async_copy(
                rows[s].at[pl.ds(b * CHUNK, CHUNK)],
                out_hbm.at[b, pl.ds(pos0 + k * CHUNK, CHUNK)],
                wsem[s],
            )
            for b in range(BATCH)
        ]
    for s in range(NSLOT):
        if wcp[s] is not None:
            for c in wcp[s]:
                c.wait()


def kernel(x, table):
    x = x.astype(jnp.int32)
    xr = x.reshape(BATCH, NW, NCHUNK, CHUNK).transpose(1, 2, 0, 3)
    xr = xr.reshape(NW, NCHUNK, ROWS)
    pe = _positional_encoding()
    return _sc_embed(xr, table, pe)


# speedup vs baseline: 3.5053x; 3.3043x over previous
import functools
import math

import jax
import jax.numpy as jnp
import numpy as np
from jax import lax
from jax.experimental import pallas as pl
from jax.experimental.pallas import tpu as pltpu
from jax.experimental.pallas import tpu_sc as plsc

VOCAB = 100000
DMODEL = 768
MAXCTX = 2048
BATCH = 4
SEQ = 2048

NUM_CORES = 2
NUM_SUBCORES = 16
NW = NUM_CORES * NUM_SUBCORES
POS_PER_W = SEQ // NW
CHUNK = 8
NCHUNK = POS_PER_W // CHUNK
ROWS = BATCH * CHUNK
NSLOT = 3
LANES = 16
VREGS_PER_ROW = DMODEL // LANES
UNROLL = 8
SCALE = math.sqrt(float(DMODEL))


def _positional_encoding_np():
    pos = np.arange(MAXCTX, dtype=np.float32)[:, None]
    i = np.arange(DMODEL, dtype=np.float32)[None, :]
    angles = 1.0 / np.power(np.float32(10000.0),
                            2.0 * np.floor(i / 2.0) / np.float32(DMODEL))
    angle_rads = (pos * angles).astype(np.float32)
    sines = np.sin(angle_rads[:, 0::2])
    cosines = np.cos(angle_rads[:, 1::2])
    return np.concatenate([sines, cosines], axis=-1).astype(np.float32)


_PE = _positional_encoding_np()


_mesh = plsc.VectorSubcoreMesh(core_axis_name="c", subcore_axis_name="s")


@functools.partial(
    pl.kernel,
    mesh=_mesh,
    out_type=jax.ShapeDtypeStruct((BATCH, SEQ, DMODEL), jnp.float32),
    scratch_types=[
        pltpu.VMEM((NCHUNK, ROWS), jnp.int32),
        pltpu.VMEM((POS_PER_W, DMODEL), jnp.float32),
        pltpu.VMEM((ROWS, DMODEL), jnp.float32),
        pltpu.VMEM((ROWS, DMODEL), jnp.float32),
        pltpu.VMEM((ROWS, DMODEL), jnp.float32),
        pltpu.SemaphoreType.DMA,
        pltpu.SemaphoreType.DMA,
        pltpu.SemaphoreType.DMA,
        pltpu.SemaphoreType.DMA,
        pltpu.SemaphoreType.DMA,
        pltpu.SemaphoreType.DMA,
    ],
)
def _sc_embed(xr_hbm, table_hbm, pe_hbm, out_hbm, idx_v, pe_v,
              rows0, rows1, rows2, g0, g1, g2, w0, w1, w2, ):
    rows = (rows0, rows1, rows2)
    gsem = (g0, g1, g2)
    wsem = (w0, w1, w2)
    w = lax.axis_index("s") * NUM_CORES + lax.axis_index("c")
    pos0 = w * POS_PER_W

    pltpu.sync_copy(xr_hbm.at[w], idx_v)
    pltpu.sync_copy(pe_hbm.at[pl.ds(pos0, POS_PER_W)], pe_v)

    def compute(rv, k):
        def body_i(i, _):
            def body_c(c4, _):
                for u in range(UNROLL):
                    start = pl.multiple_of((c4 * UNROLL + u) * LANES, LANES)
                    sl = pl.ds(start, LANES)
                    pv = pe_v[k * CHUNK + i, sl]
                    for b in range(BATCH):
                        r = rv[b * CHUNK + i, sl]
                        rv[b * CHUNK + i, sl] = r * SCALE + pv
                return 0

            lax.fori_loop(0, VREGS_PER_ROW // UNROLL, body_c, 0)
            return 0

        lax.fori_loop(0, CHUNK, body_i, 0)

    gcp = [None] * NSLOT
    wcp = [None] * NSLOT
    gcp[0] = pltpu.async_copy(table_hbm.at[idx_v.at[0]], rows[0], gsem[0])
    for k in range(NCHUNK):
        s = k % NSLOT
        if k + 1 < NCHUNK:
            ns = (k + 1) % NSLOT
            if wcp[ns] is not None:
                for c in wcp[ns]:
                    c.wait()
                wcp[ns] = None
            gcp[ns] = pltpu.async_copy(
                table_hbm.at[idx_v.at[k + 1]], rows[ns], gsem[ns])
        gcp[s].wait()
        compute(rows[s], k)
        wcp[s] = [
            pltpu.async_copy(
                rows[s].at[pl.ds(b * CHUNK, CHUNK)],
                out_hbm.at[b, pl.ds(pos0 + k * CHUNK, CHUNK)],
                wsem[s],
            )
            for b in range(BATCH)
        ]
    for s in range(NSLOT):
        if wcp[s] is not None:
            for c in wcp[s]:
                c.wait()


def kernel(x, table):
    x = x.astype(jnp.int32)
    xr = x.reshape(BATCH, NW, NCHUNK, CHUNK).transpose(1, 2, 0, 3)
    xr = xr.reshape(NW, NCHUNK, ROWS)
    pe = jnp.asarray(_PE)
    return _sc_embed(xr, table, pe)


# speedup vs baseline: 7.0983x; 2.0250x over previous
import functools
import math

import jax
import jax.numpy as jnp
import numpy as np
from jax import lax
from jax.experimental import pallas as pl
from jax.experimental.pallas import tpu as pltpu
from jax.experimental.pallas import tpu_sc as plsc

VOCAB = 100000
DMODEL = 768
MAXCTX = 2048
BATCH = 4
SEQ = 2048

NUM_CORES = 2
NUM_SUBCORES = 16
NW = NUM_CORES * NUM_SUBCORES
POS_PER_W = SEQ // NW
CHUNK = 8
NCHUNK = POS_PER_W // CHUNK
ROWS = BATCH * CHUNK
NSLOT = 3
LANES = 16
VREGS_PER_ROW = DMODEL // LANES
UNROLL = 8
SCALE = math.sqrt(float(DMODEL))


def _positional_encoding_np():
    pos = np.arange(MAXCTX, dtype=np.float32)[:, None]
    i = np.arange(DMODEL, dtype=np.float32)[None, :]
    angles = 1.0 / np.power(np.float32(10000.0),
                            2.0 * np.floor(i / 2.0) / np.float32(DMODEL))
    angle_rads = (pos * angles).astype(np.float32)
    sines = np.sin(angle_rads[:, 0::2])
    cosines = np.cos(angle_rads[:, 1::2])
    return np.concatenate([sines, cosines], axis=-1).astype(np.float32)


_PE = _positional_encoding_np()


_mesh = plsc.VectorSubcoreMesh(core_axis_name="c", subcore_axis_name="s")


@functools.partial(
    pl.kernel,
    mesh=_mesh,
    out_type=jax.ShapeDtypeStruct((BATCH, SEQ, DMODEL), jnp.float32),
    scratch_types=[
        pltpu.VMEM((NCHUNK, ROWS), jnp.int32),
        pltpu.VMEM((POS_PER_W, DMODEL), jnp.float32),
        pltpu.VMEM((ROWS, DMODEL), jnp.float32),
        pltpu.VMEM((ROWS, DMODEL), jnp.float32),
        pltpu.VMEM((ROWS, DMODEL), jnp.float32),
        pltpu.SemaphoreType.DMA,
        pltpu.SemaphoreType.DMA,
        pltpu.SemaphoreType.DMA,
        pltpu.SemaphoreType.DMA,
        pltpu.SemaphoreType.DMA,
        pltpu.SemaphoreType.DMA,
    ],
)
def _sc_embed(xr_hbm, table_hbm, pe_hbm, out_hbm, idx_v, pe_v,
              rows0, rows1, rows2, g0, g1, g2, w0, w1, w2, ):
    rows = (rows0, rows1, rows2)
    gsem = (g0, g1, g2)
    wsem = (w0, w1, w2)
    w = lax.axis_index("s") * NUM_CORES + lax.axis_index("c")
    pos0 = w * POS_PER_W

    pltpu.sync_copy(xr_hbm.at[w], idx_v)
    pltpu.sync_copy(pe_hbm.at[pl.ds(pos0, POS_PER_W)], pe_v)

    def compute(rv, k):
        @plsc.parallel_loop(0, VREGS_PER_ROW, 1, unroll=1)
        def _(c):
            sl = pl.ds(pl.multiple_of(c * LANES, LANES), LANES)
            for i in range(CHUNK):
                pv = pe_v[k * CHUNK + i, sl]
                for b in range(BATCH):
                    row = b * CHUNK + i
                    rv[row, sl] = rv[row, sl] * SCALE + pv

    gcp = [None] * NSLOT
    wcp = [None] * NSLOT
    gcp[0] = pltpu.async_copy(table_hbm.at[idx_v.at[0]], rows[0], gsem[0])
    for k in range(NCHUNK):
        s = k % NSLOT
        if k + 1 < NCHUNK:
            ns = (k + 1) % NSLOT
            if wcp[ns] is not None:
                for c in wcp[ns]:
                    c.wait()
                wcp[ns] = None
            gcp[ns] = pltpu.async_copy(
                table_hbm.at[idx_v.at[k + 1]], rows[ns], gsem[ns])
        gcp[s].wait()
        compute(rows[s], k)
        wcp[s] = [
            pltpu.async_copy(
                rows[s].at[pl.ds(b * CHUNK, CHUNK)],
                out_hbm.at[b, pl.ds(pos0 + k * CHUNK, CHUNK)],
                wsem[s],
            )
            for b in range(BATCH)
        ]
    for s in range(NSLOT):
        if wcp[s] is not None:
            for c in wcp[s]:
                c.wait()


def kernel(x, table):
    x = x.astype(jnp.int32)
    xr = x.reshape(BATCH, NW, NCHUNK, CHUNK).transpose(1, 2, 0, 3)
    xr = xr.reshape(NW, NCHUNK, ROWS)
    pe = jnp.asarray(_PE)
    return _sc_embed(xr, table, pe)


# speedup vs baseline: 7.1466x; 1.0068x over previous
import functools
import math

import jax
import jax.numpy as jnp
import numpy as np
from jax import lax
from jax.experimental import pallas as pl
from jax.experimental.pallas import tpu as pltpu
from jax.experimental.pallas import tpu_sc as plsc

VOCAB = 100000
DMODEL = 768
MAXCTX = 2048
BATCH = 4
SEQ = 2048

NUM_CORES = 2
NUM_SUBCORES = 16
NW = NUM_CORES * NUM_SUBCORES
POS_PER_W = SEQ // NW
CHUNK = 8
NCHUNK = POS_PER_W // CHUNK
ROWS = BATCH * CHUNK
NSLOT = 3
LANES = 16
VREGS_PER_ROW = DMODEL // LANES
UNROLL = 8
SCALE = math.sqrt(float(DMODEL))


def _positional_encoding_np():
    pos = np.arange(MAXCTX, dtype=np.float32)[:, None]
    i = np.arange(DMODEL, dtype=np.float32)[None, :]
    angles = 1.0 / np.power(np.float32(10000.0),
                            2.0 * np.floor(i / 2.0) / np.float32(DMODEL))
    angle_rads = (pos * angles).astype(np.float32)
    sines = np.sin(angle_rads[:, 0::2])
    cosines = np.cos(angle_rads[:, 1::2])
    return np.concatenate([sines, cosines], axis=-1).astype(np.float32)


_PE = _positional_encoding_np()


_mesh = plsc.VectorSubcoreMesh(core_axis_name="c", subcore_axis_name="s")


@functools.partial(
    pl.kernel,
    mesh=_mesh,
    out_type=jax.ShapeDtypeStruct((BATCH, SEQ, DMODEL), jnp.float32),
    scratch_types=[
        pltpu.VMEM((NCHUNK, ROWS), jnp.int32),
        pltpu.VMEM((POS_PER_W, DMODEL), jnp.float32),
        pltpu.VMEM((ROWS, DMODEL), jnp.float32),
        pltpu.VMEM((ROWS, DMODEL), jnp.float32),
        pltpu.VMEM((ROWS, DMODEL), jnp.float32),
        pltpu.SemaphoreType.DMA,
        pltpu.SemaphoreType.DMA,
        pltpu.SemaphoreType.DMA,
        pltpu.SemaphoreType.DMA,
        pltpu.SemaphoreType.DMA,
        pltpu.SemaphoreType.DMA,
        pltpu.SemaphoreType.DMA,
    ],
)
def _sc_embed(xr_hbm, table_hbm, pe_hbm, out_hbm, idx_v, pe_v,
              rows0, rows1, rows2, g0, g1, g2, w0, w1, w2, psem):
    rows = (rows0, rows1, rows2)
    gsem = (g0, g1, g2)
    wsem = (w0, w1, w2)
    w = lax.axis_index("s") * NUM_CORES + lax.axis_index("c")
    pos0 = w * POS_PER_W

    pltpu.sync_copy(xr_hbm.at[w], idx_v)
    pe_cp = pltpu.async_copy(pe_hbm.at[pl.ds(pos0, POS_PER_W)], pe_v, psem)

    def compute(rv, k):
        @plsc.parallel_loop(0, VREGS_PER_ROW, 1, unroll=1)
        def _(c):
            sl = pl.ds(pl.multiple_of(c * LANES, LANES), LANES)
            for i in range(CHUNK):
                pv = pe_v[k * CHUNK + i, sl]
                for b in range(BATCH):
                    row = b * CHUNK + i
                    rv[row, sl] = rv[row, sl] * SCALE + pv

    gcp = [None] * NSLOT
    wcp = [None] * NSLOT
    gcp[0] = pltpu.async_copy(table_hbm.at[idx_v.at[0]], rows[0], gsem[0])
    for k in range(NCHUNK):
        s = k % NSLOT
        if k + 1 < NCHUNK:
            ns = (k + 1) % NSLOT
            if wcp[ns] is not None:
                for c in wcp[ns]:
                    c.wait()
                wcp[ns] = None
            gcp[ns] = pltpu.async_copy(
                table_hbm.at[idx_v.at[k + 1]], rows[ns], gsem[ns])
        if k == 0:
            pe_cp.wait()
        gcp[s].wait()
        compute(rows[s], k)
        wcp[s] = [
            pltpu.async_copy(
                rows[s].at[pl.ds(b * CHUNK, CHUNK)],
                out_hbm.at[b, pl.ds(pos0 + k * CHUNK, CHUNK)],
                wsem[s],
            )
            for b in range(BATCH)
        ]
    for s in range(NSLOT):
        if wcp[s] is not None:
            for c in wcp[s]:
                c.wait()


def kernel(x, table):
    x = x.astype(jnp.int32)
    xr = x.reshape(BATCH, NW, NCHUNK, CHUNK).transpose(1, 2, 0, 3)
    xr = xr.reshape(NW, NCHUNK, ROWS)
    pe = jnp.asarray(_PE)
    return _sc_embed(xr, table, pe)


# speedup vs baseline: 7.2894x; 1.0200x over previous
import functools
import math

import jax
import jax.numpy as jnp
import numpy as np
from jax import lax
from jax.experimental import pallas as pl
from jax.experimental.pallas import tpu as pltpu
from jax.experimental.pallas import tpu_sc as plsc

VOCAB = 100000
DMODEL = 768
MAXCTX = 2048
BATCH = 4
SEQ = 2048

NUM_CORES = 2
NUM_SUBCORES = 16
NW = NUM_CORES * NUM_SUBCORES
POS_PER_W = SEQ // NW
CHUNK = 8
NCHUNK = POS_PER_W // CHUNK
ROWS = BATCH * CHUNK
NSLOT = 4
LANES = 16
VREGS_PER_ROW = DMODEL // LANES
SCALE = math.sqrt(float(DMODEL))


def _positional_encoding_np():
    pos = np.arange(MAXCTX, dtype=np.float32)[:, None]
    i = np.arange(DMODEL, dtype=np.float32)[None, :]
    angles = 1.0 / np.power(np.float32(10000.0),
                            2.0 * np.floor(i / 2.0) / np.float32(DMODEL))
    angle_rads = (pos * angles).astype(np.float32)
    sines = np.sin(angle_rads[:, 0::2])
    cosines = np.cos(angle_rads[:, 1::2])
    return np.concatenate([sines, cosines], axis=-1).astype(np.float32)


_PE = _positional_encoding_np()


_mesh = plsc.VectorSubcoreMesh(core_axis_name="c", subcore_axis_name="s")


@functools.partial(
    pl.kernel,
    mesh=_mesh,
    out_type=jax.ShapeDtypeStruct((BATCH, SEQ, DMODEL), jnp.float32),
    scratch_types=[
        pltpu.VMEM((NCHUNK, ROWS), jnp.int32),
        pltpu.VMEM((CHUNK, DMODEL), jnp.float32),
        pltpu.VMEM((CHUNK, DMODEL), jnp.float32),
        pltpu.VMEM((CHUNK, DMODEL), jnp.float32),
        pltpu.VMEM((CHUNK, DMODEL), jnp.float32),
        pltpu.VMEM((ROWS, DMODEL), jnp.float32),
        pltpu.VMEM((ROWS, DMODEL), jnp.float32),
        pltpu.VMEM((ROWS, DMODEL), jnp.float32),
        pltpu.VMEM((ROWS, DMODEL), jnp.float32),
        pltpu.SemaphoreType.DMA,
        pltpu.SemaphoreType.DMA,
        pltpu.SemaphoreType.DMA,
        pltpu.SemaphoreType.DMA,
        pltpu.SemaphoreType.DMA,
        pltpu.SemaphoreType.DMA,
        pltpu.SemaphoreType.DMA,
        pltpu.SemaphoreType.DMA,
    ],
)
def _sc_embed(xr_hbm, table_hbm, pe_hbm, out_hbm, idx_v,
              pe0, pe1, pe2, pe3, rows0, rows1, rows2, rows3,
              g0, g1, g2, g3, w0, w1, w2, w3):
    pes = (pe0, pe1, pe2, pe3)
    rows = (rows0, rows1, rows2, rows3)
    gsem = (g0, g1, g2, g3)
    wsem = (w0, w1, w2, w3)
    w = lax.axis_index("s") * NUM_CORES + lax.axis_index("c")
    pos0 = w * POS_PER_W

    pltpu.sync_copy(xr_hbm.at[w], idx_v)

    def start_chunk(k):
        s = k % NSLOT
        g = pltpu.async_copy(table_hbm.at[idx_v.at[k]], rows[s], gsem[s])
        p = pltpu.async_copy(
            pe_hbm.at[pl.ds(pos0 + k * CHUNK, CHUNK)], pes[s], gsem[s])
        return (g, p)

    def compute(rv, pv):
        @plsc.parallel_loop(0, VREGS_PER_ROW, 1, unroll=1)
        def _(c):
            sl = pl.ds(pl.multiple_of(c * LANES, LANES), LANES)
            for i in range(CHUNK):
                pvv = pv[i, sl]
                for b in range(BATCH):
                    row = b * CHUNK + i
                    rv[row, sl] = rv[row, sl] * SCALE + pvv

    gcp = [None] * NSLOT
    wcp = [None] * NSLOT
    gcp[0] = start_chunk(0)
    gcp[1] = start_chunk(1)
    for k in range(NCHUNK):
        s = k % NSLOT
        if k + 2 < NCHUNK:
            ns = (k + 2) % NSLOT
            if wcp[ns] is not None:
                for c in wcp[ns]:
                    c.wait()
                wcp[ns] = None
            gcp[ns] = start_chunk(k + 2)
        for c in gcp[s]:
            c.wait()
        compute(rows[s], pes[s])
        wcp[s] = [
            pltpu.async_copy(
                rows[s].at[pl.ds(b * CHUNK, CHUNK)],
                out_hbm.at[b, pl.ds(pos0 + k * CHUNK, CHUNK)],
                wsem[s],
            )
            for b in range(BATCH)
        ]
    for s in range(NSLOT):
        if wcp[s] is not None:
            for c in wcp[s]:
                c.wait()


def kernel(x, table):
    x = x.astype(jnp.int32)
    xr = x.reshape(BATCH, NW, NCHUNK, CHUNK).transpose(1, 2, 0, 3)
    xr = xr.reshape(NW, NCHUNK, ROWS)
    pe = jnp.asarray(_PE)
    return _sc_embed(xr, table, pe)


# speedup vs baseline: 7.3371x; 1.0065x over previous
import functools
import math

import jax
import jax.numpy as jnp
import numpy as np
from jax import lax
from jax.experimental import pallas as pl
from jax.experimental.pallas import tpu as pltpu
from jax.experimental.pallas import tpu_sc as plsc

VOCAB = 100000
DMODEL = 768
MAXCTX = 2048
BATCH = 4
SEQ = 2048

NUM_CORES = 2
NUM_SUBCORES = 16
NW = NUM_CORES * NUM_SUBCORES
POS_PER_W = SEQ // NW
CHUNK = 8
NCHUNK = POS_PER_W // CHUNK
ROWS = BATCH * CHUNK
NSLOT = 4
LANES = 16
VREGS_PER_ROW = DMODEL // LANES
SCALE = math.sqrt(float(DMODEL))


def _positional_encoding_np():
    pos = np.arange(MAXCTX, dtype=np.float32)[:, None]
    i = np.arange(DMODEL, dtype=np.float32)[None, :]
    angles = 1.0 / np.power(np.float32(10000.0),
                            2.0 * np.floor(i / 2.0) / np.float32(DMODEL))
    angle_rads = (pos * angles).astype(np.float32)
    sines = np.sin(angle_rads[:, 0::2])
    cosines = np.cos(angle_rads[:, 1::2])
    return np.concatenate([sines, cosines], axis=-1).astype(np.float32)


_PE = _positional_encoding_np()


_mesh = plsc.VectorSubcoreMesh(core_axis_name="c", subcore_axis_name="s")


@functools.partial(
    pl.kernel,
    mesh=_mesh,
    out_type=jax.ShapeDtypeStruct((BATCH, SEQ, DMODEL), jnp.float32),
    scratch_types=[
        pltpu.VMEM((BATCH * POS_PER_W,), jnp.int32),
        pltpu.VMEM((CHUNK, DMODEL), jnp.float32),
        pltpu.VMEM((CHUNK, DMODEL), jnp.float32),
        pltpu.VMEM((CHUNK, DMODEL), jnp.float32),
        pltpu.VMEM((CHUNK, DMODEL), jnp.float32),
        pltpu.VMEM((ROWS, DMODEL), jnp.float32),
        pltpu.VMEM((ROWS, DMODEL), jnp.float32),
        pltpu.VMEM((ROWS, DMODEL), jnp.float32),
        pltpu.VMEM((ROWS, DMODEL), jnp.float32),
        pltpu.SemaphoreType.DMA,
        pltpu.SemaphoreType.DMA,
        pltpu.SemaphoreType.DMA,
        pltpu.SemaphoreType.DMA,
        pltpu.SemaphoreType.DMA,
        pltpu.SemaphoreType.DMA,
        pltpu.SemaphoreType.DMA,
        pltpu.SemaphoreType.DMA,
    ],
)
def _sc_embed(x_hbm, table_hbm, pe_hbm, out_hbm, xraw_v,
              pe0, pe1, pe2, pe3, rows0, rows1, rows2, rows3,
              g0, g1, g2, g3, w0, w1, w2, w3):
    pes = (pe0, pe1, pe2, pe3)
    rows = (rows0, rows1, rows2, rows3)
    gsem = (g0, g1, g2, g3)
    wsem = (w0, w1, w2, w3)
    w = lax.axis_index("s") * NUM_CORES + lax.axis_index("c")
    pos0 = w * POS_PER_W

    for b in range(BATCH):
        pltpu.sync_copy(x_hbm.at[b, pl.ds(pos0, POS_PER_W)],
                        xraw_v.at[pl.ds(b * POS_PER_W, POS_PER_W)])


    def start_chunk(k):
        s = k % NSLOT
        gs = [
            pltpu.async_copy(
                table_hbm.at[xraw_v.at[pl.ds(b * POS_PER_W + k * CHUNK, CHUNK)]],
                rows[s].at[pl.ds(b * CHUNK, CHUNK)],
                gsem[s],
            )
            for b in range(BATCH)
        ]
        p = pltpu.async_copy(
            pe_hbm.at[pl.ds(pos0 + k * CHUNK, CHUNK)], pes[s], gsem[s])
        return gs + [p]

    def compute(rv, pv):
        @plsc.parallel_loop(0, VREGS_PER_ROW, 1, unroll=1)
        def _(c):
            sl = pl.ds(pl.multiple_of(c * LANES, LANES), LANES)
            for i in range(CHUNK):
                pvv = pv[i, sl]
                for b in range(BATCH):
                    row = b * CHUNK + i
                    rv[row, sl] = rv[row, sl] * SCALE + pvv

    gcp = [None] * NSLOT
    wcp = [None] * NSLOT
    gcp[0] = start_chunk(0)
    gcp[1] = start_chunk(1)
    for k in range(NCHUNK):
        s = k % NSLOT
        if k + 2 < NCHUNK:
            ns = (k + 2) % NSLOT
            if wcp[ns] is not None:
                for c in wcp[ns]:
                    c.wait()
                wcp[ns] = None
            gcp[ns] = start_chunk(k + 2)
        for c in gcp[s]:
            c.wait()
        compute(rows[s], pes[s])
        wcp[s] = [
            pltpu.async_copy(
                rows[s].at[pl.ds(b * CHUNK, CHUNK)],
                out_hbm.at[b, pl.ds(pos0 + k * CHUNK, CHUNK)],
                wsem[s],
            )
            for b in range(BATCH)
        ]
    for s in range(NSLOT):
        if wcp[s] is not None:
            for c in wcp[s]:
                c.wait()


def kernel(x, table):
    x = x.astype(jnp.int32)
    pe = jnp.asarray(_PE)
    return _sc_embed(x, table, pe)


# speedup vs baseline: 7.6129x; 1.0376x over previous
import functools
import math

import jax
import jax.numpy as jnp
import numpy as np
from jax import lax
from jax.experimental import pallas as pl
from jax.experimental.pallas import tpu as pltpu
from jax.experimental.pallas import tpu_sc as plsc

VOCAB = 100000
DMODEL = 768
MAXCTX = 2048
BATCH = 4
SEQ = 2048

NUM_CORES = 2
NUM_SUBCORES = 16
NW = NUM_CORES * NUM_SUBCORES
POS_PER_W = SEQ // NW
CHUNK = 8
NCHUNK = POS_PER_W // CHUNK
ROWS = BATCH * CHUNK
NSLOT = 4
LANES = 16
VREGS_PER_ROW = DMODEL // LANES
SCALE = math.sqrt(float(DMODEL))


def _positional_encoding_np():
    pos = np.arange(MAXCTX, dtype=np.float32)[:, None]
    i = np.arange(DMODEL, dtype=np.float32)[None, :]
    angles = 1.0 / np.power(np.float32(10000.0),
                            2.0 * np.floor(i / 2.0) / np.float32(DMODEL))
    angle_rads = (pos * angles).astype(np.float32)
    sines = np.sin(angle_rads[:, 0::2])
    cosines = np.cos(angle_rads[:, 1::2])
    return np.concatenate([sines, cosines], axis=-1).astype(np.float32)


_PE = _positional_encoding_np()


_mesh = plsc.VectorSubcoreMesh(core_axis_name="c", subcore_axis_name="s")


@functools.partial(
    pl.kernel,
    mesh=_mesh,
    out_type=jax.ShapeDtypeStruct((BATCH, SEQ, DMODEL), jnp.float32),
    scratch_types=[
        pltpu.VMEM((BATCH * POS_PER_W,), jnp.int32),
        pltpu.VMEM((CHUNK, DMODEL), jnp.float32),
        pltpu.VMEM((CHUNK, DMODEL), jnp.float32),
        pltpu.VMEM((CHUNK, DMODEL), jnp.float32),
        pltpu.VMEM((CHUNK, DMODEL), jnp.float32),
        pltpu.VMEM((ROWS, DMODEL), jnp.float32),
        pltpu.VMEM((ROWS, DMODEL), jnp.float32),
        pltpu.VMEM((ROWS, DMODEL), jnp.float32),
        pltpu.VMEM((ROWS, DMODEL), jnp.float32),
        pltpu.SemaphoreType.DMA,
        pltpu.SemaphoreType.DMA,
        pltpu.SemaphoreType.DMA,
        pltpu.SemaphoreType.DMA,
        pltpu.SemaphoreType.DMA,
        pltpu.SemaphoreType.DMA,
        pltpu.SemaphoreType.DMA,
        pltpu.SemaphoreType.DMA,
        pltpu.SemaphoreType.DMA,
    ],
)
def _sc_embed(x_hbm, table_hbm, pe_hbm, out_hbm, xraw_v,
              pe0, pe1, pe2, pe3, rows0, rows1, rows2, rows3,
              g0, g1, g2, g3, w0, w1, w2, w3, xsem):
    pes = (pe0, pe1, pe2, pe3)
    rows = (rows0, rows1, rows2, rows3)
    gsem = (g0, g1, g2, g3)
    wsem = (w0, w1, w2, w3)
    w = lax.axis_index("s") * NUM_CORES + lax.axis_index("c")
    pos0 = w * POS_PER_W

    xcps = [
        pltpu.async_copy(x_hbm.at[b, pl.ds(pos0, POS_PER_W)],
                         xraw_v.at[pl.ds(b * POS_PER_W, POS_PER_W)], xsem)
        for b in range(BATCH)
    ]


    def start_pe(k):
        s = k % NSLOT
        return pltpu.async_copy(
            pe_hbm.at[pl.ds(pos0 + k * CHUNK, CHUNK)], pes[s], gsem[s])

    def start_gathers(k):
        s = k % NSLOT
        return [
            pltpu.async_copy(
                table_hbm.at[xraw_v.at[pl.ds(b * POS_PER_W + k * CHUNK, CHUNK)]],
                rows[s].at[pl.ds(b * CHUNK, CHUNK)],
                gsem[s],
            )
            for b in range(BATCH)
        ]

    def start_chunk(k):
        return start_gathers(k) + [start_pe(k)]

    def compute(rv, pv):
        @plsc.parallel_loop(0, VREGS_PER_ROW, 1, unroll=1)
        def _(c):
            sl = pl.ds(pl.multiple_of(c * LANES, LANES), LANES)
            for i in range(CHUNK):
                pvv = pv[i, sl]
                for b in range(BATCH):
                    row = b * CHUNK + i
                    rv[row, sl] = rv[row, sl] * SCALE + pvv

    gcp = [None] * NSLOT
    wcp = [None] * NSLOT
    pe0cp = start_pe(0)
    pe1cp = start_pe(1)
    for c in xcps:
        c.wait()
    gcp[0] = start_gathers(0) + [pe0cp]
    gcp[1] = start_gathers(1) + [pe1cp]
    for k in range(NCHUNK):
        s = k % NSLOT
        if k + 2 < NCHUNK:
            ns = (k + 2) % NSLOT
            if wcp[ns] is not None:
                for c in wcp[ns]:
                    c.wait()
                wcp[ns] = None
            gcp[ns] = start_chunk(k + 2)
        for c in gcp[s]:
            c.wait()
        compute(rows[s], pes[s])
        wcp[s] = [
            pltpu.async_copy(
                rows[s].at[pl.ds(b * CHUNK, CHUNK)],
                out_hbm.at[b, pl.ds(pos0 + k * CHUNK, CHUNK)],
                wsem[s],
            )
            for b in range(BATCH)
        ]
    for s in range(NSLOT):
        if wcp[s] is not None:
            for c in wcp[s]:
                c.wait()


def kernel(x, table):
    x = x.astype(jnp.int32)
    pe = jnp.asarray(_PE)
    return _sc_embed(x, table, pe)
